# baseline (speedup 1.0000x reference)
import functools

import jax
import jax.numpy as jnp
from jax import lax
from jax.experimental import pallas as pl
from jax.experimental.pallas import tpu as pltpu
from jax.experimental.pallas import tpu_sc as plsc

_NUM_CORES = 2
_NUM_SUBCORES = 16
_NUM_WORKERS = _NUM_CORES * _NUM_SUBCORES


@functools.partial(jax.jit, static_argnames=())
def _lookup(inputs, table):
    batch = inputs.shape[0]
    b_per_w = batch // _NUM_WORKERS

    mesh = plsc.VectorSubcoreMesh(
        core_axis_name="c", subcore_axis_name="s", num_cores=_NUM_CORES
    )

    nchunk = 4
    ch = b_per_w // nchunk

    @functools.partial(
        pl.kernel,
        mesh=mesh,
        out_type=jax.ShapeDtypeStruct((batch,), jnp.int32),
        scratch_types=[
            pltpu.VMEM((b_per_w,), jnp.int32),
            pltpu.VMEM((b_per_w,), jnp.int32),
            pltpu.SemaphoreType.DMA,
            pltpu.SemaphoreType.DMA,
            pltpu.SemaphoreType.DMA,
        ],
    )
    def gather_kernel(
        inputs_hbm, table_hbm, out_hbm, idx_v, vals_v, sem_i, sem_g, sem_o
    ):
        wid = lax.axis_index("s") * _NUM_CORES + lax.axis_index("c")
        base = wid * b_per_w
        idx_h = [
            pltpu.async_copy(
                inputs_hbm.at[pl.ds(base + c * ch, ch)],
                idx_v.at[pl.ds(c * ch, ch)],
                sem_i,
            )
            for c in range(nchunk)
        ]
        g_h = []
        for c in range(nchunk):
            idx_h[c].wait()
            g_h.append(
                pltpu.async_copy(
                    table_hbm.at[idx_v.at[pl.ds(c * ch, ch)]],
                    vals_v.at[pl.ds(c * ch, ch)],
                    sem_g,
                )
            )
        o_h = []
        for c in range(nchunk):
            g_h[c].wait()
            o_h.append(
                pltpu.async_copy(
                    vals_v.at[pl.ds(c * ch, ch)],
                    out_hbm.at[pl.ds(base + c * ch, ch)],
                    sem_o,
                )
            )
        for h in o_h:
            h.wait()

    return gather_kernel(inputs, table)


def kernel(inputs, table):
    out = _lookup(inputs.astype(jnp.int32), table.astype(jnp.int32))
    return out.astype(table.dtype)


# speedup vs baseline: 1.0048x; 1.0048x over previous
import functools

import jax
import jax.numpy as jnp
from jax import lax
from jax.experimental import pallas as pl
from jax.experimental.pallas import tpu as pltpu
from jax.experimental.pallas import tpu_sc as plsc

_NUM_CORES = 2
_NUM_SUBCORES = 16
_NUM_WORKERS = _NUM_CORES * _NUM_SUBCORES


@functools.partial(jax.jit, static_argnames=())
def _lookup(inputs, table):
    batch = inputs.shape[0]
    b_per_w = batch // _NUM_WORKERS

    mesh = plsc.VectorSubcoreMesh(
        core_axis_name="c", subcore_axis_name="s", num_cores=_NUM_CORES
    )

    nchunk = 2
    ch = b_per_w // nchunk

    @functools.partial(
        pl.kernel,
        mesh=mesh,
        out_type=jax.ShapeDtypeStruct((batch,), jnp.int32),
        scratch_types=[
            pltpu.VMEM((b_per_w,), jnp.int32),
            pltpu.VMEM((b_per_w,), jnp.int32),
            pltpu.SemaphoreType.DMA,
            pltpu.SemaphoreType.DMA,
            pltpu.SemaphoreType.DMA,
        ],
    )
    def gather_kernel(
        inputs_hbm, table_hbm, out_hbm, idx_v, vals_v, sem_i, sem_g, sem_o
    ):
        wid = lax.axis_index("s") * _NUM_CORES + lax.axis_index("c")
        base = wid * b_per_w
        idx_h = [
            pltpu.async_copy(
                inputs_hbm.at[pl.ds(base + c * ch, ch)],
                idx_v.at[pl.ds(c * ch, ch)],
                sem_i,
            )
            for c in range(nchunk)
        ]
        g_h = []
        for c in range(nchunk):
            idx_h[c].wait()
            g_h.append(
                pltpu.async_copy(
                    table_hbm.at[idx_v.at[pl.ds(c * ch, ch)]],
                    vals_v.at[pl.ds(c * ch, ch)],
                    sem_g,
                )
            )
        o_h = []
        for c in range(nchunk):
            g_h[c].wait()
            o_h.append(
                pltpu.async_copy(
                    vals_v.at[pl.ds(c * ch, ch)],
                    out_hbm.at[pl.ds(base + c * ch, ch)],
                    sem_o,
                )
            )
        for h in o_h:
            h.wait()

    return gather_kernel(inputs, table)


def kernel(inputs, table):
    out = _lookup(inputs.astype(jnp.int32), table.astype(jnp.int32))
    return out.astype(table.dtype)
